# baseline (speedup 1.0000x reference)
import functools

import jax
import jax.numpy as jnp
from jax import lax
from jax.experimental import pallas as pl
from jax.experimental.pallas import tpu as pltpu
from jax.experimental.pallas import tpu_sc as plsc

N = 10000
E = 320000
TILES = 16
ROWS_PT = 624
ROWS_TAIL = N - TILES * ROWS_PT
CH = 64
NP_COL = 320
NP_EDGE = 160
E_PAD = 16 * NP_COL * CH
PHASE = 40
NBUF = 4
DH = 128


def _make_spmm(colsplit):
    npc = NP_COL if colsplit else NP_EDGE
    mesh = plsc.VectorSubcoreMesh(core_axis_name="c", subcore_axis_name="s")

    @functools.partial(
        pl.kernel,
        out_type=(
            jax.ShapeDtypeStruct((N, DH), jnp.float32),
            jax.ShapeDtypeStruct((N, DH), jnp.float32),
        ),
        mesh=mesh,
        compiler_params=pltpu.CompilerParams(use_tc_tiling_on_sc=False),
        scratch_types=[
            pltpu.VMEM((PHASE, CH), jnp.int32),
            pltpu.VMEM((PHASE, CH), jnp.int32),
            pltpu.VMEM((PHASE, CH), jnp.float32),
            pltpu.VMEM((CH, DH), jnp.float32),
            pltpu.VMEM((CH, DH), jnp.float32),
            pltpu.VMEM((CH, DH), jnp.float32),
            pltpu.VMEM((CH, DH), jnp.float32),
            pltpu.VMEM_SHARED((N, DH), jnp.float32),
            [pltpu.SemaphoreType.DMA] * 4,
            [pltpu.SemaphoreType.DMA] * 4,
        ],
    )
    def spmm(f_a, f_b, src3, dst3, w3, zeros_hbm, out_a, out_b,
             src_v, dst_v, w_v, buf0, buf1, buf2, buf3, acc, sgs, sss):
        c = lax.axis_index("c")
        s = lax.axis_index("s")
        bufs = (buf0, buf1, buf2, buf3)
        row0 = s * ROWS_PT

        pltpu.sync_copy(zeros_hbm.at[pl.ds(row0, ROWS_PT)],
                        acc.at[pl.ds(row0, ROWS_PT)])

        @pl.when(s == TILES - 1)
        def _():
            pltpu.sync_copy(zeros_hbm.at[pl.ds(TILES * ROWS_PT, ROWS_TAIL)],
                            acc.at[pl.ds(TILES * ROWS_PT, ROWS_TAIL)])

        plane = s if colsplit else TILES * c + s
        plsc.subcore_barrier()

        def run(feats, out):
            def gather(i, buf, sg):
                pltpu.async_copy(feats.at[src_v.at[i]], buf, sg)

            def gather_wait(i, buf, sg):
                pltpu.make_async_copy(feats.at[src_v.at[i]], buf, sg).wait()

            def scatter(i, buf, ss):
                pltpu.async_copy(buf, acc.at[dst_v.at[i]], ss, add=True)

            def scatter_wait(i, buf, ss):
                pltpu.make_async_copy(buf, acc.at[dst_v.at[i]], ss).wait()

            def phase_body(ph, carry):
                base = pl.multiple_of(ph * PHASE, PHASE)
                pltpu.sync_copy(src3.at[plane, pl.ds(base, PHASE)], src_v)
                pltpu.sync_copy(dst3.at[plane, pl.ds(base, PHASE)], dst_v)
                pltpu.sync_copy(w3.at[plane, pl.ds(base, PHASE)], w_v)
                gather(0, buf0, sgs[0])
                gather(1, buf1, sgs[1])

                def outer(o, carry2):
                    for b in range(NBUF):
                        i = o * NBUF + b
                        buf = bufs[b]
                        gather_wait(i, buf, sgs[b])

                        def grp(g, cc):
                            w16 = w_v[i, pl.ds(g * 16, 16)]
                            for k in range(16):
                                we = w16[k]
                                e = g * 16 + k
                                for j in range(DH // 16):
                                    sl = pl.ds(j * 16, 16)
                                    buf[e, sl] = buf[e, sl] * we
                            return cc

                        b2 = (b + 2) % NBUF

                        @pl.when(i >= 2)
                        def _():
                            scatter_wait(i - 2, bufs[b2], sss[b2])

                        @pl.when(i + 2 < PHASE)
                        def _():
                            gather(i + 2, bufs[b2], sgs[b2])

                        lax.fori_loop(0, CH // 16, grp, 0)
                        scatter(i, buf, sss[b])
                    return carry2

                lax.fori_loop(0, PHASE // NBUF, outer, 0)
                scatter_wait(PHASE - 2, bufs[(PHASE - 2) % NBUF],
                             sss[(PHASE - 2) % NBUF])
                scatter_wait(PHASE - 1, bufs[(PHASE - 1) % NBUF],
                             sss[(PHASE - 1) % NBUF])
                return carry

            lax.fori_loop(0, npc // PHASE, phase_body, 0)
            plsc.subcore_barrier()

            pltpu.sync_copy(acc.at[pl.ds(row0, ROWS_PT)],
                            out.at[pl.ds(row0, ROWS_PT)])

            @pl.when(s == TILES - 1)
            def _():
                pltpu.sync_copy(acc.at[pl.ds(TILES * ROWS_PT, ROWS_TAIL)],
                                out.at[pl.ds(TILES * ROWS_PT, ROWS_TAIL)])

        @pl.when(c == 0)
        def _():
            run(f_a, out_a)

        @pl.when(c == 1)
        def _():
            run(f_b, out_b)

    return spmm


_spmm_edge = _make_spmm(False)

def _acc_zero(zeros_hbm, acc, s):
    row0 = s * ROWS_PT
    pltpu.sync_copy(zeros_hbm.at[pl.ds(row0, ROWS_PT)],
                    acc.at[pl.ds(row0, ROWS_PT)])

    @pl.when(s == TILES - 1)
    def _():
        pltpu.sync_copy(zeros_hbm.at[pl.ds(TILES * ROWS_PT, ROWS_TAIL)],
                        acc.at[pl.ds(TILES * ROWS_PT, ROWS_TAIL)])


def _acc_write(acc, out, s):
    row0 = s * ROWS_PT
    pltpu.sync_copy(acc.at[pl.ds(row0, ROWS_PT)],
                    out.at[pl.ds(row0, ROWS_PT)])

    @pl.when(s == TILES - 1)
    def _():
        pltpu.sync_copy(acc.at[pl.ds(TILES * ROWS_PT, ROWS_TAIL)],
                        out.at[pl.ds(TILES * ROWS_PT, ROWS_TAIL)])


CHA = 32
NPA = 320
PHASEA = 16


def _make_spmm_a2():
    mesh = plsc.VectorSubcoreMesh(core_axis_name="c", subcore_axis_name="s")

    @functools.partial(
        pl.kernel,
        out_type=(
            jax.ShapeDtypeStruct((N, DH), jnp.float32),
            jax.ShapeDtypeStruct((N, DH), jnp.float32),
            jax.ShapeDtypeStruct((2 * TILES * NPA, CHA, 2 * DH),
                                 jnp.float32),
        ),
        mesh=mesh,
        compiler_params=pltpu.CompilerParams(use_tc_tiling_on_sc=False),
        scratch_types=[
            pltpu.VMEM((PHASEA, CHA), jnp.int32),
            pltpu.VMEM((PHASEA, CHA), jnp.int32),
            pltpu.VMEM((PHASEA, CHA), jnp.float32),
            pltpu.VMEM((CHA, 2 * DH), jnp.float32),
            pltpu.VMEM((CHA, 2 * DH), jnp.float32),
            pltpu.VMEM((CHA, 2 * DH), jnp.float32),
            pltpu.VMEM((CHA, 2 * DH), jnp.float32),
            pltpu.VMEM((CHA, DH), jnp.float32),
            pltpu.VMEM((CHA, DH), jnp.float32),
            pltpu.VMEM_SHARED((N, DH), jnp.float32),
            [pltpu.SemaphoreType.DMA] * 4,
            [pltpu.SemaphoreType.DMA] * 2,
            [pltpu.SemaphoreType.DMA] * 4,
        ],
    )
    def spmm_a(feats, src3, dst3, w3, zeros_hbm, part_lo, part_hi, msg,
               src_v, dst_v, w_v, g0, g1, g2, g3, st0, st1, acc,
               sgs, sss, sws):
        c = lax.axis_index("c")
        s = lax.axis_index("s")
        gbufs = (g0, g1, g2, g3)
        sts = (st0, st1)

        _acc_zero(zeros_hbm, acc, s)
        plane = TILES * c + s
        plsc.subcore_barrier()

        def run(off, part):
            def phase_body(ph, carry):
                base = pl.multiple_of(ph * PHASEA, PHASEA)
                pltpu.sync_copy(src3.at[plane, pl.ds(base, PHASEA)], src_v)
                pltpu.sync_copy(dst3.at[plane, pl.ds(base, PHASEA)], dst_v)
                pltpu.sync_copy(w3.at[plane, pl.ds(base, PHASEA)], w_v)
                qbase = plane * NPA + base
                pltpu.async_copy(feats.at[src_v.at[0]], g0, sgs[0])
                pltpu.async_copy(feats.at[src_v.at[1]], g1, sgs[1])

                def outer(o, carry2):
                    for b in range(4):
                        i = o * 4 + b
                        gb = gbufs[b]
                        stw = b % 2
                        st = sts[stw]
                        pltpu.make_async_copy(
                            feats.at[src_v.at[i]], gb, sgs[b]).wait()
                        b2 = (b + 2) % 4

                        @pl.when(i >= 2)
                        def _():
                            pltpu.make_async_copy(
                                gbufs[b2], msg.at[qbase + i - 2],
                                sws[b2]).wait()
                            pltpu.make_async_copy(
                                st, acc.at[dst_v.at[i - 2]],
                                sss[stw]).wait()

                        @pl.when(i + 2 < PHASEA)
                        def _():
                            pltpu.async_copy(
                                feats.at[src_v.at[i + 2]], gbufs[b2],
                                sgs[b2])

                        def grp(g, cc):
                            w16 = w_v[i, pl.ds(g * 16, 16)]
                            for k in range(16):
                                we = w16[k]
                                e = g * 16 + k
                                for j in range(DH // 16):
                                    st[e, pl.ds(j * 16, 16)] = (
                                        gb[e, pl.ds(off + j * 16, 16)] * we)
                            return cc

                        lax.fori_loop(0, CHA // 16, grp, 0)
                        pltpu.async_copy(st, acc.at[dst_v.at[i]],
                                         sss[stw], add=True)
                        pltpu.async_copy(gb, msg.at[qbase + i], sws[b])
                    return carry2

                lax.fori_loop(0, PHASEA // 4, outer, 0)
                for i in (PHASEA - 2, PHASEA - 1):
                    pltpu.make_async_copy(
                        gbufs[i % 4], msg.at[qbase + i], sws[i % 4]).wait()
                    pltpu.make_async_copy(
                        sts[i % 2], acc.at[dst_v.at[i]], sss[i % 2]).wait()
                return carry

            lax.fori_loop(0, NPA // PHASEA, phase_body, 0)
            plsc.subcore_barrier()
            _acc_write(acc, part, s)

        @pl.when(c == 0)
        def _():
            run(0, part_lo)

        @pl.when(c == 1)
        def _():
            run(DH, part_hi)

    return spmm_a


def _make_spmm_b2():
    mesh = plsc.VectorSubcoreMesh(core_axis_name="c", subcore_axis_name="s")

    @functools.partial(
        pl.kernel,
        out_type=(
            jax.ShapeDtypeStruct((N, DH), jnp.float32),
            jax.ShapeDtypeStruct((N, DH), jnp.float32),
        ),
        mesh=mesh,
        compiler_params=pltpu.CompilerParams(use_tc_tiling_on_sc=False),
        scratch_types=[
            pltpu.VMEM((PHASEA, CHA), jnp.int32),
            pltpu.VMEM((PHASEA, CHA), jnp.float32),
            pltpu.VMEM((CHA, 2 * DH), jnp.float32),
            pltpu.VMEM((CHA, 2 * DH), jnp.float32),
            pltpu.VMEM((CHA, 2 * DH), jnp.float32),
            pltpu.VMEM((CHA, 2 * DH), jnp.float32),
            pltpu.VMEM((CHA, DH), jnp.float32),
            pltpu.VMEM((CHA, DH), jnp.float32),
            pltpu.VMEM_SHARED((N, DH), jnp.float32),
            [pltpu.SemaphoreType.DMA] * 4,
            [pltpu.SemaphoreType.DMA] * 2,
        ],
    )
    def spmm_b(part_lo, part_hi, msg, dst3, w3, out_lo, out_hi,
               dst_v, w_v, l0, l1, l2, l3, st0, st1, acc, sls, sss):
        c = lax.axis_index("c")
        s = lax.axis_index("s")
        lbufs = (l0, l1, l2, l3)
        sts = (st0, st1)
        row0 = s * ROWS_PT
        plane = TILES * (1 - c) + s
        q0 = plane * NPA

        def run(off, part, out):
            pltpu.sync_copy(part.at[pl.ds(row0, ROWS_PT)],
                            acc.at[pl.ds(row0, ROWS_PT)])

            @pl.when(s == TILES - 1)
            def _():
                pltpu.sync_copy(part.at[pl.ds(TILES * ROWS_PT, ROWS_TAIL)],
                                acc.at[pl.ds(TILES * ROWS_PT, ROWS_TAIL)])

            plsc.subcore_barrier()

            def phase_body(ph, carry):
                base = pl.multiple_of(ph * PHASEA, PHASEA)
                pltpu.sync_copy(dst3.at[plane, pl.ds(base, PHASEA)], dst_v)
                pltpu.sync_copy(w3.at[plane, pl.ds(base, PHASEA)], w_v)
                qbase = q0 + base
                pltpu.async_copy(msg.at[qbase + 0], l0, sls[0])
                pltpu.async_copy(msg.at[qbase + 1], l1, sls[1])

                def outer(o, carry2):
                    for b in range(4):
                        i = o * 4 + b
                        lb = lbufs[b]
                        stw = b % 2
                        st = sts[stw]
                        pltpu.make_async_copy(
                            msg.at[qbase + i], lb, sls[b]).wait()
                        b2 = (b + 2) % 4

                        @pl.when(i + 2 < PHASEA)
                        def _():
                            pltpu.async_copy(msg.at[qbase + i + 2],
                                             lbufs[b2], sls[b2])

                        @pl.when(i >= 2)
                        def _():
                            pltpu.make_async_copy(
                                st, acc.at[dst_v.at[i - 2]],
                                sss[stw]).wait()

                        def grp(g, cc):
                            w16 = w_v[i, pl.ds(g * 16, 16)]
                            for k in range(16):
                                we = w16[k]
                                e = g * 16 + k
                                for j in range(DH // 16):
                                    st[e, pl.ds(j * 16, 16)] = (
                                        lb[e, pl.ds(off + j * 16, 16)] * we)
                            return cc

                        lax.fori_loop(0, CHA // 16, grp, 0)
                        pltpu.async_copy(st, acc.at[dst_v.at[i]],
                                         sss[stw], add=True)
                    return carry2

                lax.fori_loop(0, PHASEA // 4, outer, 0)
                for i in (PHASEA - 2, PHASEA - 1):
                    pltpu.make_async_copy(
                        sts[i % 2], acc.at[dst_v.at[i]], sss[i % 2]).wait()
                return carry

            lax.fori_loop(0, NPA // PHASEA, phase_body, 0)
            plsc.subcore_barrier()
            _acc_write(acc, out, s)

        @pl.when(c == 0)
        def _():
            run(0, part_lo, out_lo)

        @pl.when(c == 1)
        def _():
            run(DH, part_hi, out_hi)

    return spmm_b


_spmm_a2 = _make_spmm_a2()
_spmm_b2 = _make_spmm_b2()


_ROW_BLK = 1000


def _dense1_body(z_ref, w_ref, out_ref):
    out_ref[...] = jnp.tanh(jnp.dot(z_ref[...], w_ref[...],
                                    preferred_element_type=jnp.float32))


def _dense1(z, w):
    dout = w.shape[1]
    return pl.pallas_call(
        _dense1_body,
        grid=(N // _ROW_BLK,),
        in_specs=[
            pl.BlockSpec((_ROW_BLK, z.shape[1]), lambda i: (i, 0)),
            pl.BlockSpec((w.shape[0], dout), lambda i: (0, 0)),
        ],
        out_specs=pl.BlockSpec((_ROW_BLK, dout), lambda i: (i, 0)),
        out_shape=jax.ShapeDtypeStruct((N, dout), jnp.float32),
    )(z, w)


def _dense2_body(xlo_ref, xhi_ref, wt_ref, wb_ref, lo_ref, hi_ref):
    y = jnp.dot(xlo_ref[...], wt_ref[...], preferred_element_type=jnp.float32)
    y = y + jnp.dot(xhi_ref[...], wb_ref[...],
                    preferred_element_type=jnp.float32)
    y = jnp.tanh(y)
    d = y.shape[1] // 2
    lo_ref[...] = y[:, :d]
    hi_ref[...] = y[:, d:]


def _dense2(xlo, xhi, w):
    k = xlo.shape[1]
    dout = w.shape[1]
    d = dout // 2
    wt, wb = w[:k], w[k:]
    return pl.pallas_call(
        _dense2_body,
        grid=(N // _ROW_BLK,),
        in_specs=[
            pl.BlockSpec((_ROW_BLK, k), lambda i: (i, 0)),
            pl.BlockSpec((_ROW_BLK, k), lambda i: (i, 0)),
            pl.BlockSpec((k, dout), lambda i: (0, 0)),
            pl.BlockSpec((k, dout), lambda i: (0, 0)),
        ],
        out_specs=[
            pl.BlockSpec((_ROW_BLK, d), lambda i: (i, 0)),
            pl.BlockSpec((_ROW_BLK, d), lambda i: (i, 0)),
        ],
        out_shape=[
            jax.ShapeDtypeStruct((N, d), jnp.float32),
            jax.ShapeDtypeStruct((N, d), jnp.float32),
        ],
    )(xlo, xhi, wt, wb)


def _dense3_body(xlo_ref, xhi_ref, wt_ref, wb_ref, out_ref):
    y = jnp.dot(xlo_ref[...], wt_ref[...], preferred_element_type=jnp.float32)
    y = y + jnp.dot(xhi_ref[...], wb_ref[...],
                    preferred_element_type=jnp.float32)
    out_ref[...] = jnp.tanh(y)


def _dense3(xlo, xhi, w):
    k = xlo.shape[1]
    dout = w.shape[1]
    wt, wb = w[:k], w[k:]
    return pl.pallas_call(
        _dense3_body,
        grid=(N // _ROW_BLK,),
        in_specs=[
            pl.BlockSpec((_ROW_BLK, k), lambda i: (i, 0)),
            pl.BlockSpec((_ROW_BLK, k), lambda i: (i, 0)),
            pl.BlockSpec((k, dout), lambda i: (0, 0)),
            pl.BlockSpec((k, dout), lambda i: (0, 0)),
        ],
        out_specs=pl.BlockSpec((_ROW_BLK, dout), lambda i: (i, 0)),
        out_shape=jax.ShapeDtypeStruct((N, dout), jnp.float32),
    )(xlo, xhi, wt, wb)


_RB = 2000
_CB = 2048


def _recon_body(p0r_ref, p1r_ref, p0c_ref, p1c_ref, h_ref, out_ref):
    hr = p0r_ref[...] + p1r_ref[...]
    hc = p0c_ref[...] + p1c_ref[...]
    h_ref[...] = hr
    z = lax.dot_general(hr.astype(jnp.bfloat16), hc.astype(jnp.bfloat16),
                        (((1,), (1,)), ((), ())),
                        preferred_element_type=jnp.float32)
    out_ref[...] = 0.5 * jnp.tanh(0.5 * z) + 0.5


def _recon(p0, p1):
    d = p0.shape[1]
    return pl.pallas_call(
        _recon_body,
        grid=(N // _RB, pl.cdiv(N, _CB)),
        in_specs=[
            pl.BlockSpec((_RB, d), lambda i, j: (i, 0)),
            pl.BlockSpec((_RB, d), lambda i, j: (i, 0)),
            pl.BlockSpec((_CB, d), lambda i, j: (j, 0)),
            pl.BlockSpec((_CB, d), lambda i, j: (j, 0)),
        ],
        out_specs=[
            pl.BlockSpec((_RB, d), lambda i, j: (i, 0)),
            pl.BlockSpec((_RB, _CB), lambda i, j: (i, j)),
        ],
        out_shape=[
            jax.ShapeDtypeStruct((N, d), jnp.float32),
            jax.ShapeDtypeStruct((N, N), jnp.float32),
        ],
    )(p0, p1, p0, p1)


def kernel(z_igae, edge_index, edge_weight, W4, W5, W6):
    pad = E_PAD - E
    src = jnp.pad(edge_index[1], (0, pad))
    dst = jnp.pad(edge_index[0], (0, pad))
    w = jnp.pad(edge_weight, (0, pad))
    src_a, dst_a, w_a = (x.reshape(2 * TILES, NPA, CHA)
                         for x in (src, dst, w))
    src_b, dst_b, w_b = (x.reshape(2 * TILES, NP_EDGE, CH)
                         for x in (src, dst, w))
    zeros128 = jnp.zeros((N, DH), jnp.float32)

    h1 = _dense1(z_igae, W4)
    pa, pb, msg = _spmm_a2(h1, src_a, dst_a, w_a, zeros128)
    s1_lo, s1_hi = _spmm_b2(pa, pb, msg, dst_a, w_a)
    h2 = _dense3(s1_lo, s1_hi, W5)
    pa2, pb2, msg2 = _spmm_a2(h2, src_a, dst_a, w_a, zeros128)
    s2_lo, s2_hi = _spmm_b2(pa2, pb2, msg2, dst_a, w_a)
    h3 = _dense3(s2_lo, s2_hi, W6)
    p0, p1 = _spmm_edge(h3, h3, src_b, dst_b, w_b, zeros128)
    h, adj_rec = _recon(p0, p1)
    return (h, adj_rec)


# speedup vs baseline: 1.9850x; 1.9850x over previous
import functools

import jax
import jax.numpy as jnp
from jax import lax
from jax.experimental import pallas as pl
from jax.experimental.pallas import tpu as pltpu
from jax.experimental.pallas import tpu_sc as plsc

N = 10000
E = 320000
TILES = 16
ROWS_PT = 624
ROWS_TAIL = N - TILES * ROWS_PT
CH = 64
NP_COL = 320
NP_EDGE = 160
E_PAD = 16 * NP_COL * CH
PHASE = 40
NBUF = 4
DH = 128


def _make_spmm(colsplit):
    npc = NP_COL if colsplit else NP_EDGE
    mesh = plsc.VectorSubcoreMesh(core_axis_name="c", subcore_axis_name="s")

    @functools.partial(
        pl.kernel,
        out_type=(
            jax.ShapeDtypeStruct((N, DH), jnp.float32),
            jax.ShapeDtypeStruct((N, DH), jnp.float32),
        ),
        mesh=mesh,
        compiler_params=pltpu.CompilerParams(use_tc_tiling_on_sc=False),
        scratch_types=[
            pltpu.VMEM((PHASE, CH), jnp.int32),
            pltpu.VMEM((PHASE, CH), jnp.int32),
            pltpu.VMEM((PHASE, CH), jnp.float32),
            pltpu.VMEM((CH, DH), jnp.float32),
            pltpu.VMEM((CH, DH), jnp.float32),
            pltpu.VMEM((CH, DH), jnp.float32),
            pltpu.VMEM((CH, DH), jnp.float32),
            pltpu.VMEM_SHARED((N, DH), jnp.float32),
            [pltpu.SemaphoreType.DMA] * 4,
            [pltpu.SemaphoreType.DMA] * 4,
        ],
    )
    def spmm(f_a, f_b, src3, dst3, w3, zeros_hbm, out_a, out_b,
             src_v, dst_v, w_v, buf0, buf1, buf2, buf3, acc, sgs, sss):
        c = lax.axis_index("c")
        s = lax.axis_index("s")
        bufs = (buf0, buf1, buf2, buf3)
        row0 = s * ROWS_PT

        pltpu.sync_copy(zeros_hbm.at[pl.ds(row0, ROWS_PT)],
                        acc.at[pl.ds(row0, ROWS_PT)])

        @pl.when(s == TILES - 1)
        def _():
            pltpu.sync_copy(zeros_hbm.at[pl.ds(TILES * ROWS_PT, ROWS_TAIL)],
                            acc.at[pl.ds(TILES * ROWS_PT, ROWS_TAIL)])

        plane = s if colsplit else TILES * c + s
        plsc.subcore_barrier()

        def run(feats, out):
            def gather(i, buf, sg):
                pltpu.async_copy(feats.at[src_v.at[i]], buf, sg)

            def gather_wait(i, buf, sg):
                pltpu.make_async_copy(feats.at[src_v.at[i]], buf, sg).wait()

            def scatter(i, buf, ss):
                pltpu.async_copy(buf, acc.at[dst_v.at[i]], ss, add=True)

            def scatter_wait(i, buf, ss):
                pltpu.make_async_copy(buf, acc.at[dst_v.at[i]], ss).wait()

            def phase_body(ph, carry):
                base = pl.multiple_of(ph * PHASE, PHASE)
                pltpu.sync_copy(src3.at[plane, pl.ds(base, PHASE)], src_v)
                pltpu.sync_copy(dst3.at[plane, pl.ds(base, PHASE)], dst_v)
                pltpu.sync_copy(w3.at[plane, pl.ds(base, PHASE)], w_v)
                gather(0, buf0, sgs[0])
                gather(1, buf1, sgs[1])

                def outer(o, carry2):
                    for b in range(NBUF):
                        i = o * NBUF + b
                        buf = bufs[b]
                        gather_wait(i, buf, sgs[b])

                        def grp(g, cc):
                            w16 = w_v[i, pl.ds(g * 16, 16)]
                            for k in range(16):
                                we = w16[k]
                                e = g * 16 + k
                                for j in range(DH // 16):
                                    sl = pl.ds(j * 16, 16)
                                    buf[e, sl] = buf[e, sl] * we
                            return cc

                        b2 = (b + 2) % NBUF

                        @pl.when(i >= 2)
                        def _():
                            scatter_wait(i - 2, bufs[b2], sss[b2])

                        @pl.when(i + 2 < PHASE)
                        def _():
                            gather(i + 2, bufs[b2], sgs[b2])

                        lax.fori_loop(0, CH // 16, grp, 0)
                        scatter(i, buf, sss[b])
                    return carry2

                lax.fori_loop(0, PHASE // NBUF, outer, 0)
                scatter_wait(PHASE - 2, bufs[(PHASE - 2) % NBUF],
                             sss[(PHASE - 2) % NBUF])
                scatter_wait(PHASE - 1, bufs[(PHASE - 1) % NBUF],
                             sss[(PHASE - 1) % NBUF])
                return carry

            lax.fori_loop(0, npc // PHASE, phase_body, 0)
            plsc.subcore_barrier()

            pltpu.sync_copy(acc.at[pl.ds(row0, ROWS_PT)],
                            out.at[pl.ds(row0, ROWS_PT)])

            @pl.when(s == TILES - 1)
            def _():
                pltpu.sync_copy(acc.at[pl.ds(TILES * ROWS_PT, ROWS_TAIL)],
                                out.at[pl.ds(TILES * ROWS_PT, ROWS_TAIL)])

        @pl.when(c == 0)
        def _():
            run(f_a, out_a)

        @pl.when(c == 1)
        def _():
            run(f_b, out_b)

    return spmm


_spmm_col = _make_spmm(True)
_spmm_edge = _make_spmm(False)


_ROW_BLK = 1000


def _dense1_body(z_ref, w_ref, lo_ref, hi_ref):
    y = jnp.tanh(jnp.dot(z_ref[...], w_ref[...],
                         preferred_element_type=jnp.float32))
    d = y.shape[1] // 2
    lo_ref[...] = y[:, :d]
    hi_ref[...] = y[:, d:]


def _dense1(z, w):
    dout = w.shape[1]
    d = dout // 2
    return pl.pallas_call(
        _dense1_body,
        grid=(N // _ROW_BLK,),
        in_specs=[
            pl.BlockSpec((_ROW_BLK, z.shape[1]), lambda i: (i, 0)),
            pl.BlockSpec((w.shape[0], dout), lambda i: (0, 0)),
        ],
        out_specs=[
            pl.BlockSpec((_ROW_BLK, d), lambda i: (i, 0)),
            pl.BlockSpec((_ROW_BLK, d), lambda i: (i, 0)),
        ],
        out_shape=[
            jax.ShapeDtypeStruct((N, d), jnp.float32),
            jax.ShapeDtypeStruct((N, d), jnp.float32),
        ],
    )(z, w)


def _dense2_body(xlo_ref, xhi_ref, wt_ref, wb_ref, lo_ref, hi_ref):
    y = jnp.dot(xlo_ref[...], wt_ref[...], preferred_element_type=jnp.float32)
    y = y + jnp.dot(xhi_ref[...], wb_ref[...],
                    preferred_element_type=jnp.float32)
    y = jnp.tanh(y)
    d = y.shape[1] // 2
    lo_ref[...] = y[:, :d]
    hi_ref[...] = y[:, d:]


def _dense2(xlo, xhi, w):
    k = xlo.shape[1]
    dout = w.shape[1]
    d = dout // 2
    wt, wb = w[:k], w[k:]
    return pl.pallas_call(
        _dense2_body,
        grid=(N // _ROW_BLK,),
        in_specs=[
            pl.BlockSpec((_ROW_BLK, k), lambda i: (i, 0)),
            pl.BlockSpec((_ROW_BLK, k), lambda i: (i, 0)),
            pl.BlockSpec((k, dout), lambda i: (0, 0)),
            pl.BlockSpec((k, dout), lambda i: (0, 0)),
        ],
        out_specs=[
            pl.BlockSpec((_ROW_BLK, d), lambda i: (i, 0)),
            pl.BlockSpec((_ROW_BLK, d), lambda i: (i, 0)),
        ],
        out_shape=[
            jax.ShapeDtypeStruct((N, d), jnp.float32),
            jax.ShapeDtypeStruct((N, d), jnp.float32),
        ],
    )(xlo, xhi, wt, wb)


def _dense3_body(xlo_ref, xhi_ref, wt_ref, wb_ref, out_ref):
    y = jnp.dot(xlo_ref[...], wt_ref[...], preferred_element_type=jnp.float32)
    y = y + jnp.dot(xhi_ref[...], wb_ref[...],
                    preferred_element_type=jnp.float32)
    out_ref[...] = jnp.tanh(y)


def _dense3(xlo, xhi, w):
    k = xlo.shape[1]
    dout = w.shape[1]
    wt, wb = w[:k], w[k:]
    return pl.pallas_call(
        _dense3_body,
        grid=(N // _ROW_BLK,),
        in_specs=[
            pl.BlockSpec((_ROW_BLK, k), lambda i: (i, 0)),
            pl.BlockSpec((_ROW_BLK, k), lambda i: (i, 0)),
            pl.BlockSpec((k, dout), lambda i: (0, 0)),
            pl.BlockSpec((k, dout), lambda i: (0, 0)),
        ],
        out_specs=pl.BlockSpec((_ROW_BLK, dout), lambda i: (i, 0)),
        out_shape=jax.ShapeDtypeStruct((N, dout), jnp.float32),
    )(xlo, xhi, wt, wb)


_RB = 2000
_CB = 2048


def _recon_body(p0r_ref, p1r_ref, p0c_ref, p1c_ref, h_ref, out_ref):
    hr = p0r_ref[...] + p1r_ref[...]
    hc = p0c_ref[...] + p1c_ref[...]
    h_ref[...] = hr
    z = lax.dot_general(hr.astype(jnp.bfloat16), hc.astype(jnp.bfloat16),
                        (((1,), (1,)), ((), ())),
                        preferred_element_type=jnp.float32)
    out_ref[...] = 0.5 * jnp.tanh(0.5 * z) + 0.5


def _recon(p0, p1):
    d = p0.shape[1]
    return pl.pallas_call(
        _recon_body,
        grid=(N // _RB, pl.cdiv(N, _CB)),
        in_specs=[
            pl.BlockSpec((_RB, d), lambda i, j: (i, 0)),
            pl.BlockSpec((_RB, d), lambda i, j: (i, 0)),
            pl.BlockSpec((_CB, d), lambda i, j: (j, 0)),
            pl.BlockSpec((_CB, d), lambda i, j: (j, 0)),
        ],
        out_specs=[
            pl.BlockSpec((_RB, d), lambda i, j: (i, 0)),
            pl.BlockSpec((_RB, _CB), lambda i, j: (i, j)),
        ],
        out_shape=[
            jax.ShapeDtypeStruct((N, d), jnp.float32),
            jax.ShapeDtypeStruct((N, N), jnp.float32),
        ],
    )(p0, p1, p0, p1)


def kernel(z_igae, edge_index, edge_weight, W4, W5, W6):
    pad = E_PAD - E
    src = jnp.pad(edge_index[1], (0, pad))
    dst = jnp.pad(edge_index[0], (0, pad))
    w = jnp.pad(edge_weight, (0, pad))
    src_a, dst_a, w_a = (x.reshape(TILES, NP_COL, CH) for x in (src, dst, w))
    src_b, dst_b, w_b = (x.reshape(2 * TILES, NP_EDGE, CH)
                         for x in (src, dst, w))
    zeros128 = jnp.zeros((N, DH), jnp.float32)

    h1_lo, h1_hi = _dense1(z_igae, W4)
    s1_lo, s1_hi = _spmm_col(h1_lo, h1_hi, src_a, dst_a, w_a, zeros128)
    h2_lo, h2_hi = _dense2(s1_lo, s1_hi, W5)
    s2_lo, s2_hi = _spmm_col(h2_lo, h2_hi, src_a, dst_a, w_a, zeros128)
    h3 = _dense3(s2_lo, s2_hi, W6)
    p0, p1 = _spmm_edge(h3, h3, src_b, dst_b, w_b, zeros128)
    h, adj_rec = _recon(p0, p1)
    return (h, adj_rec)
